# Initial kernel scaffold; baseline (speedup 1.0000x reference)
import jax
import jax.numpy as jnp
from jax.experimental import pallas as pl


def kernel(x, W):
    raise NotImplementedError("write your pallas kernel here")


# speedup vs baseline: 2.1962x; 2.1962x over previous
import jax
import jax.numpy as jnp
from jax.experimental import pallas as pl
from jax.experimental.pallas import tpu as pltpu

HID = 4096
E = 64
K = 8
TB = 512


def _gating_block(x_ref, w_ref, gates_ref, idx_ref, loss_ref, counts_ref):
    i = pl.program_id(0)
    nsteps = pl.num_programs(0)

    @pl.when(i == 0)
    def _init():
        counts_ref[...] = jnp.zeros_like(counts_ref)

    x = x_ref[...].astype(jnp.bfloat16)
    w = w_ref[...].astype(jnp.bfloat16)
    logits = jax.lax.dot_general(
        x, w, (((1,), (1,)), ((), ())),
        preferred_element_type=jnp.float32)

    rowmax = jnp.max(logits, axis=-1, keepdims=True)
    z = jnp.sum(jnp.exp(logits - rowmax), axis=-1, keepdims=True)

    cols = jax.lax.broadcasted_iota(jnp.int32, (TB, E), 1)
    neg = jnp.float32(-jnp.inf)
    work = logits
    vals, idxs = [], []
    for _ in range(K):
        m = jnp.max(work, axis=-1, keepdims=True)
        a = jnp.min(jnp.where(work == m, cols, E), axis=-1, keepdims=True)
        vals.append(m)
        idxs.append(a)
        work = jnp.where(cols == a, neg, work)
    mvals = jnp.concatenate(vals, axis=-1)
    midx = jnp.concatenate(idxs, axis=-1)

    probs = jnp.exp(mvals - rowmax) / z
    denom = jnp.sum(probs, axis=-1, keepdims=True) + 1e-8
    gates_ref[...] = probs / denom
    idx_ref[...] = midx

    sel = (work == neg).astype(jnp.float32)
    counts_ref[...] += jnp.sum(sel, axis=0, keepdims=True)

    @pl.when(i == nsteps - 1)
    def _loss():
        counts = counts_ref[...]
        total = jnp.sum(counts, axis=1, keepdims=True)
        usage = counts / total
        mean_u = jnp.sum(usage, axis=1, keepdims=True) / E
        var_u = jnp.sum((usage - mean_u) ** 2, axis=1, keepdims=True) / (E - 1)
        loss_ref[...] = (var_u / (mean_u + 1e-8)) ** 2


def kernel(x, W):
    B_, S_, H_ = x.shape
    T = B_ * S_
    xf = x.reshape(T, H_)
    gates, idx, loss = pl.pallas_call(
        _gating_block,
        grid=(T // TB,),
        in_specs=[
            pl.BlockSpec((TB, H_), lambda i: (i, 0)),
            pl.BlockSpec((E, H_), lambda i: (0, 0)),
        ],
        out_specs=[
            pl.BlockSpec((TB, K), lambda i: (i, 0)),
            pl.BlockSpec((TB, K), lambda i: (i, 0)),
            pl.BlockSpec((1, 1), lambda i: (0, 0)),
        ],
        out_shape=[
            jax.ShapeDtypeStruct((T, K), jnp.float32),
            jax.ShapeDtypeStruct((T, K), jnp.int32),
            jax.ShapeDtypeStruct((1, 1), jnp.float32),
        ],
        scratch_shapes=[pltpu.VMEM((1, E), jnp.float32)],
        compiler_params=pltpu.CompilerParams(
            dimension_semantics=("arbitrary",)),
    )(xf, W)
    return (gates.reshape(B_, S_, K), idx.reshape(B_, S_, K), loss[0, 0])


# speedup vs baseline: 3.2101x; 1.4616x over previous
import jax
import jax.numpy as jnp
from jax.experimental import pallas as pl
from jax.experimental.pallas import tpu as pltpu

HID = 4096
E = 64
K = 8
TB = 512


def _gating_block(x_ref, w_ref, gates_ref, idx_ref, loss_ref,
                  counts_ref, mval_ref, midx_ref):
    i = pl.program_id(0)
    nsteps = pl.num_programs(0)

    @pl.when(i == 0)
    def _init():
        counts_ref[...] = jnp.zeros_like(counts_ref)

    x = x_ref[...]
    w = w_ref[...]
    work = jax.lax.dot_general(
        w, x, (((1,), (1,)), ((), ())),
        preferred_element_type=jnp.float32,
        precision=jax.lax.Precision.DEFAULT)

    rows = jax.lax.broadcasted_iota(jnp.int32, (E, TB), 0)
    neg = jnp.float32(-jnp.inf)
    for k in range(K):
        m = jnp.max(work, axis=0, keepdims=True)
        a = jnp.min(jnp.where(work == m, rows, E), axis=0, keepdims=True)
        mval_ref[k:k + 1, :] = m
        midx_ref[k:k + 1, :] = a
        work = jnp.where(rows == a, neg, work)

    rowmax = mval_ref[0:1, :]
    sel_exp = jnp.exp(mval_ref[...] - rowmax)
    rest = jnp.sum(jnp.exp(work - rowmax), axis=0, keepdims=True)
    sel_sum = jnp.sum(sel_exp, axis=0, keepdims=True)
    z = sel_sum + rest
    gates_ref[...] = (sel_exp / z) / (sel_sum / z + 1e-8)
    idx_ref[...] = midx_ref[...]

    sel = (work == neg).astype(jnp.float32)
    counts_ref[...] += jnp.sum(sel, axis=1, keepdims=True)

    @pl.when(i == nsteps - 1)
    def _loss():
        counts = counts_ref[...]
        total = jnp.sum(counts, axis=0, keepdims=True)
        usage = counts / total
        mean_u = jnp.sum(usage, axis=0, keepdims=True) / E
        var_u = jnp.sum((usage - mean_u) ** 2, axis=0, keepdims=True) / (E - 1)
        loss_ref[...] = (var_u / (mean_u + 1e-8)) ** 2


def kernel(x, W):
    B_, S_, H_ = x.shape
    T = B_ * S_
    xf = x.reshape(T, H_)
    gates_kt, idx_kt, loss = pl.pallas_call(
        _gating_block,
        grid=(T // TB,),
        in_specs=[
            pl.BlockSpec((TB, H_), lambda i: (i, 0)),
            pl.BlockSpec((E, H_), lambda i: (0, 0)),
        ],
        out_specs=[
            pl.BlockSpec((K, TB), lambda i: (0, i)),
            pl.BlockSpec((K, TB), lambda i: (0, i)),
            pl.BlockSpec((1, 1), lambda i: (0, 0)),
        ],
        out_shape=[
            jax.ShapeDtypeStruct((K, T), jnp.float32),
            jax.ShapeDtypeStruct((K, T), jnp.int32),
            jax.ShapeDtypeStruct((1, 1), jnp.float32),
        ],
        scratch_shapes=[
            pltpu.VMEM((E, 1), jnp.float32),
            pltpu.VMEM((K, TB), jnp.float32),
            pltpu.VMEM((K, TB), jnp.int32),
        ],
        compiler_params=pltpu.CompilerParams(
            dimension_semantics=("arbitrary",)),
    )(xf, W)
    gates = jnp.transpose(gates_kt).reshape(B_, S_, K)
    idx = jnp.transpose(idx_kt).reshape(B_, S_, K)
    return (gates, idx, loss[0, 0])


# speedup vs baseline: 3.4755x; 1.0827x over previous
import jax
import jax.numpy as jnp
from jax.experimental import pallas as pl
from jax.experimental.pallas import tpu as pltpu

HID = 4096
E = 64
K = 8
TB = 1024


def _gating_block(x_ref, w_ref, gates_ref, idx_ref, loss_ref,
                  counts_ref, mval_ref, midx_ref):
    i = pl.program_id(0)
    nsteps = pl.num_programs(0)

    @pl.when(i == 0)
    def _init():
        counts_ref[...] = jnp.zeros_like(counts_ref)

    x = x_ref[...]
    w = w_ref[...]
    work = jax.lax.dot_general(
        w, x, (((1,), (1,)), ((), ())),
        preferred_element_type=jnp.float32,
        precision=jax.lax.Precision.DEFAULT)

    rows = jax.lax.broadcasted_iota(jnp.int32, (E, TB), 0)
    neg = jnp.float32(-jnp.inf)
    for k in range(K):
        m = jnp.max(work, axis=0, keepdims=True)
        a = jnp.min(jnp.where(work == m, rows, E), axis=0, keepdims=True)
        mval_ref[k:k + 1, :] = m
        midx_ref[k:k + 1, :] = a
        work = jnp.where(rows == a, neg, work)

    rowmax = mval_ref[0:1, :]
    sel_exp = jnp.exp(mval_ref[...] - rowmax)
    rest = jnp.sum(jnp.exp(work - rowmax), axis=0, keepdims=True)
    sel_sum = jnp.sum(sel_exp, axis=0, keepdims=True)
    z = sel_sum + rest
    gates_ref[...] = (sel_exp / z) / (sel_sum / z + 1e-8)
    idx_ref[...] = midx_ref[...]

    sel = (work == neg).astype(jnp.float32)
    counts_ref[...] += jnp.sum(sel, axis=1, keepdims=True)

    @pl.when(i == nsteps - 1)
    def _loss():
        counts = counts_ref[...]
        total = jnp.sum(counts, axis=0, keepdims=True)
        usage = counts / total
        mean_u = jnp.sum(usage, axis=0, keepdims=True) / E
        var_u = jnp.sum((usage - mean_u) ** 2, axis=0, keepdims=True) / (E - 1)
        loss_ref[...] = (var_u / (mean_u + 1e-8)) ** 2


def kernel(x, W):
    B_, S_, H_ = x.shape
    T = B_ * S_
    xf = x.reshape(T, H_)
    gates_kt, idx_kt, loss = pl.pallas_call(
        _gating_block,
        grid=(T // TB,),
        in_specs=[
            pl.BlockSpec((TB, H_), lambda i: (i, 0)),
            pl.BlockSpec((E, H_), lambda i: (0, 0)),
        ],
        out_specs=[
            pl.BlockSpec((K, TB), lambda i: (0, i)),
            pl.BlockSpec((K, TB), lambda i: (0, i)),
            pl.BlockSpec((1, 1), lambda i: (0, 0)),
        ],
        out_shape=[
            jax.ShapeDtypeStruct((K, T), jnp.float32),
            jax.ShapeDtypeStruct((K, T), jnp.int32),
            jax.ShapeDtypeStruct((1, 1), jnp.float32),
        ],
        scratch_shapes=[
            pltpu.VMEM((E, 1), jnp.float32),
            pltpu.VMEM((K, TB), jnp.float32),
            pltpu.VMEM((K, TB), jnp.int32),
        ],
        compiler_params=pltpu.CompilerParams(
            dimension_semantics=("arbitrary",)),
    )(xf, W)
    gates = jnp.transpose(gates_kt).reshape(B_, S_, K)
    idx = jnp.transpose(idx_kt).reshape(B_, S_, K)
    return (gates, idx, loss[0, 0])
